# baseline (speedup 1.0000x reference)
import functools

import jax
import jax.numpy as jnp
from jax import lax
from jax.experimental import pallas as pl
from jax.experimental.pallas import tpu as pltpu
from jax.experimental.pallas import tpu_sc as plsc

_V = 1000000
_D = 64
_B = 4096
_S = 200

_BLK = 25000


def _rowdot_body(tbl_ref, w_ref, b_ref, out_ref):
    x = tbl_ref[...]
    w = w_ref[...]
    acc = jnp.sum(x * w, axis=1, keepdims=True)
    out_ref[...] = (acc + b_ref[0]) * (1.0 / _S)


def _rowdot(table, W, b):
    return pl.pallas_call(
        _rowdot_body,
        grid=(_V // _BLK,),
        in_specs=[
            pl.BlockSpec((_BLK, _D), lambda i: (i, 0)),
            pl.BlockSpec((1, _D), lambda i: (0, 0)),
            pl.BlockSpec(memory_space=pltpu.SMEM),
        ],
        out_specs=pl.BlockSpec((_BLK, 1), lambda i: (i, 0)),
        out_shape=jax.ShapeDtypeStruct((_V, 1), jnp.float32),
    )(table, W, b)


def _make_gather_kernel():
    info = plsc.get_sparse_core_info()
    nc, ns = info.num_cores, info.num_subcores
    nw = nc * ns
    rows_per_w = _B // nw
    idx_per_w = rows_per_w * _S
    n_grp = rows_per_w // 16

    mesh = plsc.VectorSubcoreMesh(core_axis_name="c", subcore_axis_name="s")

    @functools.partial(
        pl.kernel,
        out_type=jax.ShapeDtypeStruct((_B,), jnp.float32),
        mesh=mesh,
        scratch_types=[
            pltpu.VMEM((idx_per_w,), jnp.int32),
            pltpu.VMEM((idx_per_w,), jnp.float32),
            pltpu.SemaphoreType.DMA,
        ],
    )
    def gather_reduce(xt_hbm, t_hbm, out_hbm, idx_v, vals_v, sem):
        wid = lax.axis_index("s") * nc + lax.axis_index("c")
        base = wid * idx_per_w
        pltpu.sync_copy(xt_hbm.at[pl.ds(base, idx_per_w)], idx_v)
        pltpu.async_copy(t_hbm.at[idx_v], vals_v, sem).wait()

        def body(c, accs):
            off = c * rows_per_w
            return tuple(
                accs[g] + vals_v[pl.ds(off + g * 16, 16)]
                for g in range(n_grp)
            )

        accs = lax.fori_loop(
            0, _S, body,
            tuple(jnp.zeros((16,), jnp.float32) for _ in range(n_grp)))
        for g in range(n_grp):
            y = 1.0 / (1.0 + jnp.exp(-accs[g]))
            vals_v[pl.ds(g * 16, 16)] = y

        pltpu.sync_copy(vals_v.at[pl.ds(0, rows_per_w)],
                        out_hbm.at[pl.ds(wid * rows_per_w, rows_per_w)])

    return gather_reduce


def kernel(x, table, W, b):
    t = _rowdot(table, W, b)
    gk = _make_gather_kernel()
    nw = 32
    rows_per_w = _B // nw
    xt = x.reshape(nw, rows_per_w, _S).transpose(0, 2, 1).reshape(-1)
    out = gk(xt, t.reshape(-1))
    return out.reshape(_B, 1)


# speedup vs baseline: 1.1591x; 1.1591x over previous
import functools

import jax
import jax.numpy as jnp
from jax import lax
from jax.experimental import pallas as pl
from jax.experimental.pallas import tpu as pltpu
from jax.experimental.pallas import tpu_sc as plsc

_V = 1000000
_D = 64
_B = 4096
_S = 200

_ROWS = _V // _D
_RBLK = 512


def _rowdot_body(tbl_ref, wblk_ref, b_ref, out_ref):
    x = tbl_ref[...]
    acc = jnp.dot(x, wblk_ref[...], preferred_element_type=jnp.float32)
    out_ref[...] = (acc + b_ref[0]) * (1.0 / _S)


def _rowdot(table, W, b):
    tr = table.reshape(_ROWS, _D * _D)
    wblk = jnp.kron(jnp.eye(_D, dtype=jnp.float32), W.reshape(_D, 1))
    return pl.pallas_call(
        _rowdot_body,
        grid=(pl.cdiv(_ROWS, _RBLK),),
        in_specs=[
            pl.BlockSpec((_RBLK, _D * _D), lambda i: (i, 0)),
            pl.BlockSpec((_D * _D, _D), lambda i: (0, 0)),
            pl.BlockSpec(memory_space=pltpu.SMEM),
        ],
        out_specs=pl.BlockSpec((_RBLK, _D), lambda i: (i, 0)),
        out_shape=jax.ShapeDtypeStruct((_ROWS, _D), jnp.float32),
    )(tr, wblk, b)


def _make_gather_kernel():
    info = plsc.get_sparse_core_info()
    nc, ns = info.num_cores, info.num_subcores
    nw = nc * ns
    rows_per_w = _B // nw
    idx_per_w = rows_per_w * _S
    n_grp = rows_per_w // 16

    mesh = plsc.VectorSubcoreMesh(core_axis_name="c", subcore_axis_name="s")

    @functools.partial(
        pl.kernel,
        out_type=jax.ShapeDtypeStruct((_B,), jnp.float32),
        mesh=mesh,
        scratch_types=[
            pltpu.VMEM((idx_per_w,), jnp.int32),
            pltpu.VMEM((idx_per_w,), jnp.float32),
            pltpu.SemaphoreType.DMA,
        ],
    )
    def gather_reduce(xt_hbm, t_hbm, out_hbm, idx_v, vals_v, sem):
        wid = lax.axis_index("s") * nc + lax.axis_index("c")
        base = wid * idx_per_w
        pltpu.sync_copy(xt_hbm.at[pl.ds(base, idx_per_w)], idx_v)
        pltpu.async_copy(t_hbm.at[idx_v], vals_v, sem).wait()

        def body(c, accs):
            off = c * rows_per_w
            return tuple(
                accs[g] + vals_v[pl.ds(off + g * 16, 16)]
                for g in range(n_grp)
            )

        accs = lax.fori_loop(
            0, _S, body,
            tuple(jnp.zeros((16,), jnp.float32) for _ in range(n_grp)))
        for g in range(n_grp):
            y = 1.0 / (1.0 + jnp.exp(-accs[g]))
            vals_v[pl.ds(g * 16, 16)] = y

        pltpu.sync_copy(vals_v.at[pl.ds(0, rows_per_w)],
                        out_hbm.at[pl.ds(wid * rows_per_w, rows_per_w)])

    return gather_reduce


def kernel(x, table, W, b):
    t = _rowdot(table, W, b)
    gk = _make_gather_kernel()
    nw = 32
    rows_per_w = _B // nw
    xt = x.reshape(nw, rows_per_w, _S).transpose(0, 2, 1).reshape(-1)
    out = gk(xt, t.reshape(-1))
    return out.reshape(_B, 1)


# speedup vs baseline: 1.1633x; 1.0036x over previous
import functools

import jax
import jax.numpy as jnp
from jax import lax
from jax.experimental import pallas as pl
from jax.experimental.pallas import tpu as pltpu
from jax.experimental.pallas import tpu_sc as plsc

_V = 1000000
_D = 64
_B = 4096
_S = 200

_ROWS = _V // _D
_RBLK = 512
_NSTEP = pl.cdiv(_ROWS, _RBLK)
_FLAT = _RBLK * _D


def _rowdot_body(tbl_ref, wblk_ref, b_ref, out_ref):
    x = tbl_ref[...]
    acc = lax.dot_general(wblk_ref[...], x, (((1,), (1,)), ((), ())),
                          preferred_element_type=jnp.float32)
    out_ref[...] = ((acc + b_ref[0]) * (1.0 / _S))[None]


def _rowdot(table, W, b):
    tr = table.reshape(_ROWS, _D * _D)
    wblk = jnp.kron(jnp.eye(_D, dtype=jnp.float32), W.reshape(1, _D))
    out = pl.pallas_call(
        _rowdot_body,
        grid=(_NSTEP,),
        in_specs=[
            pl.BlockSpec((_RBLK, _D * _D), lambda i: (i, 0)),
            pl.BlockSpec((_D, _D * _D), lambda i: (0, 0)),
            pl.BlockSpec(memory_space=pltpu.SMEM),
        ],
        out_specs=pl.BlockSpec((1, _D, _RBLK), lambda i: (i, 0, 0)),
        out_shape=jax.ShapeDtypeStruct((_NSTEP, _D, _RBLK), jnp.float32),
    )(tr, wblk, b)
    return out.reshape(-1)


def _make_gather_kernel():
    info = plsc.get_sparse_core_info()
    nc, ns = info.num_cores, info.num_subcores
    nw = nc * ns
    rows_per_w = _B // nw
    idx_per_w = rows_per_w * _S
    n_grp = rows_per_w // 16
    nvr = 2 * _S // 16

    mesh = plsc.VectorSubcoreMesh(core_axis_name="c", subcore_axis_name="s")

    @functools.partial(
        pl.kernel,
        out_type=jax.ShapeDtypeStruct((_B,), jnp.float32),
        mesh=mesh,
        scratch_types=[
            pltpu.VMEM((idx_per_w,), jnp.int32),
            pltpu.VMEM((idx_per_w,), jnp.float32),
            pltpu.VMEM((rows_per_w,), jnp.float32),
            pltpu.SemaphoreType.DMA,
        ],
    )
    def gather_reduce(xt_hbm, t_hbm, out_hbm, idx_v, vals_v, out_v, sem):
        wid = lax.axis_index("s") * nc + lax.axis_index("c")
        base = wid * idx_per_w
        pltpu.sync_copy(xt_hbm.at[pl.ds(base, idx_per_w)], idx_v)
        pltpu.async_copy(t_hbm.at[idx_v], vals_v, sem).wait()

        def body(c, accs):
            off = c * rows_per_w
            return tuple(
                accs[g] + vals_v[pl.ds(off + g * 16, 16)]
                for g in range(n_grp)
            )

        accs = lax.fori_loop(
            0, _S, body,
            tuple(jnp.zeros((16,), jnp.float32) for _ in range(n_grp)))
        for g in range(n_grp):
            y = 1.0 / (1.0 + jnp.exp(-accs[g]))
            out_v[pl.ds(g * 16, 16)] = y

        pltpu.sync_copy(out_v,
                        out_hbm.at[pl.ds(wid * rows_per_w, rows_per_w)])

    return gather_reduce


def _transpose_body(x_ref, out_ref):
    v = x_ref[...]
    u = jnp.bitwise_and(v, _FLAT - 1)
    addr = (v - u) + jnp.left_shift(jnp.bitwise_and(v, _D - 1), 9) \
        + jnp.right_shift(u, 6)
    out_ref[...] = addr.T[None]


def _transpose_idx(x):
    nw = 32
    rpw = _B // nw
    return pl.pallas_call(
        _transpose_body,
        grid=(nw,),
        in_specs=[pl.BlockSpec((rpw, _S), lambda i: (i, 0))],
        out_specs=pl.BlockSpec((1, _S, rpw), lambda i: (i, 0, 0)),
        out_shape=jax.ShapeDtypeStruct((nw, _S, rpw), jnp.int32),
    )(x)


def kernel(x, table, W, b):
    t = _rowdot(table, W, b)
    xt = _transpose_idx(x)
    gk = _make_gather_kernel()
    out = gk(xt.reshape(-1), t)
    return out.reshape(_B, 1)


# speedup vs baseline: 1.5348x; 1.3194x over previous
import functools

import jax
import jax.numpy as jnp
from jax import lax
from jax.experimental import pallas as pl
from jax.experimental.pallas import tpu as pltpu
from jax.experimental.pallas import tpu_sc as plsc

_V = 1000000
_D = 64
_B = 4096
_S = 200

_VBLK = 16384
_NSTEP = pl.cdiv(_V, _VBLK)


def _rowdot_body(tbl_ref, w_ref, b_ref, out_ref):
    x = tbl_ref[...]
    acc = lax.dot_general(w_ref[...], x, (((1,), (1,)), ((), ())),
                          preferred_element_type=jnp.float32)
    out_ref[...] = ((acc[0:1] + b_ref[0]) * (1.0 / _S))[None]


def _rowdot(table, W, b):
    w8 = jnp.broadcast_to(W.reshape(1, _D), (8, _D))
    out = pl.pallas_call(
        _rowdot_body,
        grid=(_NSTEP,),
        in_specs=[
            pl.BlockSpec((_VBLK, _D), lambda i: (i, 0)),
            pl.BlockSpec((8, _D), lambda i: (0, 0)),
            pl.BlockSpec(memory_space=pltpu.SMEM),
        ],
        out_specs=pl.BlockSpec((1, 1, _VBLK), lambda i: (i, 0, 0)),
        out_shape=jax.ShapeDtypeStruct((_NSTEP, 1, _VBLK), jnp.float32),
    )(table, w8, b)
    return out.reshape(-1)


def _make_gather_kernel():
    info = plsc.get_sparse_core_info()
    nc, ns = info.num_cores, info.num_subcores
    nw = nc * ns
    rows_per_w = _B // nw
    idx_per_w = rows_per_w * _S
    n_grp = rows_per_w // 16
    nvr = 2 * _S // 16

    mesh = plsc.VectorSubcoreMesh(core_axis_name="c", subcore_axis_name="s")

    @functools.partial(
        pl.kernel,
        out_type=jax.ShapeDtypeStruct((_B,), jnp.float32),
        mesh=mesh,
        scratch_types=[
            pltpu.VMEM((idx_per_w,), jnp.int32),
            pltpu.VMEM((idx_per_w,), jnp.float32),
            pltpu.VMEM((rows_per_w,), jnp.float32),
            pltpu.SemaphoreType.DMA,
        ],
    )
    def gather_reduce(xt_hbm, t_hbm, out_hbm, idx_v, vals_v, out_v, sem):
        wid = lax.axis_index("s") * nc + lax.axis_index("c")
        base = wid * idx_per_w
        pltpu.sync_copy(xt_hbm.at[pl.ds(base, idx_per_w)], idx_v)
        pltpu.async_copy(t_hbm.at[idx_v], vals_v, sem).wait()

        def body(c, accs):
            off = c * rows_per_w
            return tuple(
                accs[g] + vals_v[pl.ds(off + g * 16, 16)]
                for g in range(n_grp)
            )

        accs = lax.fori_loop(
            0, _S, body,
            tuple(jnp.zeros((16,), jnp.float32) for _ in range(n_grp)))
        for g in range(n_grp):
            y = 1.0 / (1.0 + jnp.exp(-accs[g]))
            out_v[pl.ds(g * 16, 16)] = y

        pltpu.sync_copy(out_v,
                        out_hbm.at[pl.ds(wid * rows_per_w, rows_per_w)])

    return gather_reduce


def _transpose_body(x_ref, out_ref):
    out_ref[...] = x_ref[...].T[None]


def _transpose_idx(x):
    nw = 32
    rpw = _B // nw
    return pl.pallas_call(
        _transpose_body,
        grid=(nw,),
        in_specs=[pl.BlockSpec((rpw, _S), lambda i: (i, 0))],
        out_specs=pl.BlockSpec((1, _S, rpw), lambda i: (i, 0, 0)),
        out_shape=jax.ShapeDtypeStruct((nw, _S, rpw), jnp.int32),
    )(x)


def kernel(x, table, W, b):
    t = _rowdot(table, W, b)
    xt = _transpose_idx(x)
    gk = _make_gather_kernel()
    out = gk(xt.reshape(-1), t)
    return out.reshape(_B, 1)
